# Initial kernel scaffold; baseline (speedup 1.0000x reference)
import jax
import jax.numpy as jnp
from jax.experimental import pallas as pl


def kernel(input, mask, shift_sz, stride, triple_w, flag):
    raise NotImplementedError("write your pallas kernel here")


# speedup vs baseline: 1.0516x; 1.0516x over previous
import functools

import jax
import jax.numpy as jnp
from jax.experimental import pallas as pl
from jax.experimental.pallas import tpu as pltpu

_EPS = 1e-8
_NEG = -1e30


def _attn_body(q_ref, kv_ref, flag_ref, qflag_ref, o_ref, *, bq):
    q = q_ref[0]
    kv = kv_ref[0]
    flags = flag_ref[0]

    qn = q / (jnp.sqrt(jnp.sum(q * q, axis=-1, keepdims=True)) + _EPS)
    kn = kv / (jnp.sqrt(jnp.sum(kv * kv, axis=-1, keepdims=True)) + _EPS)

    scores = jax.lax.dot_general(
        qn, kn, (((1,), (1,)), ((), ())),
        preferred_element_type=jnp.float32)
    scores = jnp.where(flags[None, :] > 0.5, _NEG, scores)

    m = jnp.max(scores, axis=-1, keepdims=True)
    e = jnp.exp(scores - m)
    s = jnp.sum(e, axis=-1, keepdims=True)
    attn = e / s

    out = jax.lax.dot_general(
        attn, kv, (((1,), (0,)), ((), ())),
        preferred_element_type=jnp.float32)

    o_ref[0] = out * qflag_ref[0][:, None]


def _shift_attention(former_t, latter_t, flag_f, *, bq):
    B, HW, c = former_t.shape
    grid = (B, HW // bq)
    return pl.pallas_call(
        functools.partial(_attn_body, bq=bq),
        grid=grid,
        in_specs=[
            pl.BlockSpec((1, bq, c), lambda b, i: (b, i, 0)),
            pl.BlockSpec((1, HW, c), lambda b, i: (b, 0, 0)),
            pl.BlockSpec((1, HW), lambda b, i: (0, 0)),
            pl.BlockSpec((1, bq), lambda b, i: (0, i)),
        ],
        out_specs=pl.BlockSpec((1, bq, c), lambda b, i: (b, i, 0)),
        out_shape=jax.ShapeDtypeStruct((B, HW, c), jnp.float32),
        compiler_params=pltpu.CompilerParams(
            dimension_semantics=("arbitrary", "arbitrary"),
        ),
    )(former_t, latter_t, flag_f, flag_f)


def kernel(input, mask, shift_sz, stride, triple_w, flag):
    B, C, H, W = input.shape
    c = C // 2
    HW = H * W
    former = input[:, :c].reshape(B, c, HW)
    latter = input[:, c:].reshape(B, c, HW)
    former_t = former.transpose(0, 2, 1)
    latter_t = latter.transpose(0, 2, 1)
    flag_f = flag.astype(jnp.float32).reshape(1, HW)

    shift = _shift_attention(former_t, latter_t, flag_f, bq=min(512, HW))
    shift_r = shift.transpose(0, 2, 1).reshape(B, c, H, W)
    return jnp.concatenate([input, shift_r], axis=1)


# speedup vs baseline: 1.6940x; 1.6108x over previous
import functools

import jax
import jax.numpy as jnp
from jax.experimental import pallas as pl
from jax.experimental.pallas import tpu as pltpu

_EPS = 1e-8
_NEG = -1e30


def _attn_body(fm_ref, lt_ref, bias_ref, qflag_ref, o_ref, kn_ref, *, bq, c):
    i = pl.program_id(1)

    @pl.when(i == 0)
    def _init_kn():
        lt = lt_ref[0]
        kn_ref[...] = lt / (jnp.sqrt(jnp.sum(lt * lt, axis=0, keepdims=True)) + _EPS)

    fm = fm_ref[0]
    qn = fm / (jnp.sqrt(jnp.sum(fm * fm, axis=0, keepdims=True)) + _EPS)

    scores = jax.lax.dot_general(
        qn, kn_ref[...], (((0,), (0,)), ((), ())),
        preferred_element_type=jnp.float32)
    p = jnp.exp(scores + bias_ref[0][None, :])
    s = jnp.sum(p, axis=1)

    out_t = jax.lax.dot_general(
        lt_ref[0], p, (((1,), (1,)), ((), ())),
        preferred_element_type=jnp.float32)
    scale = qflag_ref[0] / s
    o_ref[0, 2 * c:, :] = out_t * scale[None, :]
    o_ref[0, :c, :] = fm
    o_ref[0, c:2 * c, :] = lt_ref[0, :, pl.ds(i * bq, bq)]


def _shift_concat(inp_chw, bias, flag_f, *, bq):
    B, C, HW = inp_chw.shape
    c = C // 2
    grid = (B, HW // bq)
    return pl.pallas_call(
        functools.partial(_attn_body, bq=bq, c=c),
        grid=grid,
        in_specs=[
            pl.BlockSpec((1, c, bq), lambda b, i: (b, 0, i)),
            pl.BlockSpec((1, c, HW), lambda b, i: (b, 1, 0)),
            pl.BlockSpec((1, HW), lambda b, i: (0, 0)),
            pl.BlockSpec((1, bq), lambda b, i: (0, i)),
        ],
        out_specs=pl.BlockSpec((1, 3 * c, bq), lambda b, i: (b, 0, i)),
        out_shape=jax.ShapeDtypeStruct((B, 3 * c, HW), jnp.float32),
        scratch_shapes=[pltpu.VMEM((c, HW), jnp.float32)],
        compiler_params=pltpu.CompilerParams(
            dimension_semantics=("arbitrary", "arbitrary"),
        ),
    )(inp_chw, inp_chw, bias, flag_f)


def kernel(input, mask, shift_sz, stride, triple_w, flag):
    B, C, H, W = input.shape
    HW = H * W
    flag_f = flag.astype(jnp.float32).reshape(1, HW)
    bias = flag_f * _NEG
    out = _shift_concat(input.reshape(B, C, HW), bias, flag_f, bq=min(512, HW))
    return out.reshape(B, C + C // 2, H, W)
